# baseline (speedup 1.0000x reference)
import jax
import jax.numpy as jnp
from jax import lax
from jax.experimental import pallas as pl
from jax.experimental.pallas import tpu as pltpu
from jax.experimental.pallas import tpu_sc as plsc

_N = 50000
_E = 800000
_HID = 32
_NP = 51200
_EP = 802816
_EW = _EP // 32
_SCH = 3584
_NSC = _EW // _SCH
_CH2 = 1792
_NS2 = _EW // _CH2
_NROWS = _NP // 16
_ER = _EP // 64


def _edge_proj_body(ea_ref, p_ref, tu_ref):
    tu_ref[...] = jnp.dot(ea_ref[...], p_ref[...],
                          preferred_element_type=jnp.float32)


def _edge_proj(ea64, p64):
    blk = 112
    return pl.pallas_call(
        _edge_proj_body,
        grid=(_ER // blk,),
        in_specs=[pl.BlockSpec((blk, 1024), lambda i: (i, 0)),
                  pl.BlockSpec((1024, 128), lambda i: (0, 0))],
        out_specs=pl.BlockSpec((blk, 128), lambda i: (i, 0)),
        out_shape=jax.ShapeDtypeStruct((_ER, 128), jnp.float32),
    )(ea64, p64)


def _scb_body(row_h, col_h, tu_h, x_h, cs_h, cd_h, acc_h,
              x_v, r0, r1, c0, c1, t0, t1, exb, exrb, tvb, idxb, ones_v,
              zbuf, csv, cdv, acc0, acc1, acc2, acc3, sem0, sem1, sems):
    cid = lax.axis_index("c")
    tid = lax.axis_index("s")
    base = (cid * 16 + tid) * _EW
    pltpu.sync_copy(x_h, x_v)
    pltpu.sync_copy(cs_h, csv)
    pltpu.sync_copy(cd_h, cdv)
    zeros16 = jnp.zeros((16,), jnp.float32)
    ones16 = jnp.ones((16,), jnp.float32)

    def zfill(i, carry):
        zbuf[pl.ds(i * 16, 16)] = zeros16
        return carry

    lax.fori_loop(0, _NROWS // 16, zfill, 0)
    for g in range(8):
        ones_v[pl.ds(g * 16, 16)] = ones16
    nbase = tid * _NROWS
    accs = (acc0, acc1, acc2, acc3)
    for a in accs:
        pltpu.sync_copy(zbuf, a.at[pl.ds(nbase, _NROWS)])
    plsc.subcore_barrier()

    cs = csv[...]
    cd = cdv[...]
    bufs = ((r0, c0, t0), (r1, c1, t1))
    lsems = (sem0, sem1)

    def start(sidx, par):
        off = base + sidx * _SCH
        return (
            pltpu.async_copy(row_h.at[pl.ds(off, _SCH)], bufs[par][0],
                             lsems[par]),
            pltpu.async_copy(col_h.at[pl.ds(off, _SCH)], bufs[par][1],
                             lsems[par]),
            pltpu.async_copy(tu_h.at[pl.ds(2 * off, 2 * _SCH)], bufs[par][2],
                             lsems[par]),
        )

    pend = {0: start(0, 0)}
    for s in range(_NSC):
        par = s % 2
        for d in pend.pop(s):
            d.wait()
        if s + 1 < _NSC:
            pend[s + 1] = start(s + 1, (s + 1) % 2)
        rb, cb, tb = bufs[par]

        def chunk(cc, carry):
            o = cc * 128
            for g in range(8):
                go = o + g * 16
                r16 = rb[pl.ds(go, 16)]
                c16 = cb[pl.ds(go, 16)]
                t16 = tb[pl.ds(((go >> 6) << 7) + (go & 63), 16)]
                xr = plsc.load_gather(x_v, [r16])
                xc = plsc.load_gather(x_v, [c16])
                al = cs * xr + cd * xc + t16
                al = jnp.where(al >= 0.0, al, 0.2 * al)
                ex = jnp.exp(al)
                lo = g * 16
                tvb[pl.ds(lo, 16)] = t16
                exb[pl.ds(lo, 16)] = ex
                exrb[pl.ds(lo, 16)] = ex * xr
                idxb[pl.ds(lo, 16)] = c16
            ds_ = (
                pltpu.async_copy(ones_v, acc0.at[idxb], sems, add=True),
                pltpu.async_copy(tvb, acc1.at[idxb], sems, add=True),
                pltpu.async_copy(exb, acc2.at[idxb], sems, add=True),
                pltpu.async_copy(exrb, acc3.at[idxb], sems, add=True),
            )
            for d in ds_:
                d.wait()
            return carry

        lax.fori_loop(0, _SCH // 128, chunk, 0)

    plsc.subcore_barrier()
    for q in range(4):
        pltpu.sync_copy(accs[q].at[pl.ds(nbase, _NROWS)],
                        acc_h.at[pl.ds((cid * 4 + q) * _NP + nbase, _NROWS)])


_scb_call = pl.kernel(
    _scb_body,
    out_type=jax.ShapeDtypeStruct((8 * _NP,), jnp.float32),
    mesh=plsc.VectorSubcoreMesh(core_axis_name="c", subcore_axis_name="s"),
    compiler_params=pltpu.CompilerParams(use_tc_tiling_on_sc=False,
                                         needs_layout_passes=False),
    scratch_types=[
        pltpu.VMEM((_NP,), jnp.float32),
        pltpu.VMEM((_SCH,), jnp.int32), pltpu.VMEM((_SCH,), jnp.int32),
        pltpu.VMEM((_SCH,), jnp.int32), pltpu.VMEM((_SCH,), jnp.int32),
        pltpu.VMEM((2 * _SCH,), jnp.float32),
        pltpu.VMEM((2 * _SCH,), jnp.float32),
        pltpu.VMEM((128,), jnp.float32), pltpu.VMEM((128,), jnp.float32),
        pltpu.VMEM((128,), jnp.float32),
        pltpu.VMEM((128,), jnp.int32), pltpu.VMEM((128,), jnp.float32),
        pltpu.VMEM((_NROWS,), jnp.float32),
        pltpu.VMEM((16,), jnp.float32), pltpu.VMEM((16,), jnp.float32),
        pltpu.VMEM_SHARED((_NP,), jnp.float32),
        pltpu.VMEM_SHARED((_NP,), jnp.float32),
        pltpu.VMEM_SHARED((_NP,), jnp.float32),
        pltpu.VMEM_SHARED((_NP,), jnp.float32),
        pltpu.SemaphoreType.DMA, pltpu.SemaphoreType.DMA,
        pltpu.SemaphoreType.DMA,
    ],
)


def _node_body(p_ref, acc_ref, x_ref, wsb_ref, wg_ref, g1_ref, g2_ref):
    csd = p_ref[0, 0]
    a = acc_ref[0] + acc_ref[1]
    deg = a[0]
    st = a[1]
    den = a[2]
    num = a[3]
    xs = x_ref[...]
    sl = st / jnp.maximum(deg, 1.0)
    als = csd * xs + sl
    als = jnp.where(als >= 0.0, als, 0.2 * als)
    exs = jnp.exp(als)
    s = (num + exs * xs) / (den + exs + 1e-16)
    h = jnp.maximum(s[:, None] * wsb_ref[0][None, :] + wsb_ref[1][None, :],
                    0.0)
    g = jnp.dot(h, wg_ref[...], preferred_element_type=jnp.float32)
    g1_ref[...] = g[:, 0]
    g2_ref[...] = g[:, 1]


def _node_pass(csd, acc, xp, wsb, wg):
    blk = 5120
    return pl.pallas_call(
        _node_body,
        grid=(_NP // blk,),
        in_specs=[
            pl.BlockSpec(memory_space=pltpu.SMEM),
            pl.BlockSpec((2, 4, blk), lambda i: (0, 0, i)),
            pl.BlockSpec((blk,), lambda i: (i,)),
            pl.BlockSpec((2, _HID), lambda i: (0, 0)),
            pl.BlockSpec((_HID, 2), lambda i: (0, 0)),
        ],
        out_specs=[pl.BlockSpec((blk,), lambda i: (i,)),
                   pl.BlockSpec((blk,), lambda i: (i,))],
        out_shape=[jax.ShapeDtypeStruct((_NP,), jnp.float32),
                   jax.ShapeDtypeStruct((_NP,), jnp.float32)],
    )(csd, acc, xp, wsb, wg)


def _scc_body(row_h, col_h, tu_h, g1_h, g2_h, bm_h, out_h,
              g1_v, g2_v, bmv, r0, r1, c0, c1, u0, u1, o0, o1,
              seml0, seml1, sems0, sems1):
    cid = lax.axis_index("c")
    tid = lax.axis_index("s")
    base = (cid * 16 + tid) * _EW
    pltpu.sync_copy(g1_h, g1_v)
    pltpu.sync_copy(g2_h, g2_v)
    pltpu.sync_copy(bm_h, bmv)
    bm = bmv[...]
    inbufs = ((r0, c0, u0), (r1, c1, u1))
    obufs = (o0, o1)
    lsems = (seml0, seml1)
    ssems = (sems0, sems1)

    def startl(sidx, par):
        off = base + sidx * _CH2
        return (
            pltpu.async_copy(row_h.at[pl.ds(off, _CH2)], inbufs[par][0],
                             lsems[par]),
            pltpu.async_copy(col_h.at[pl.ds(off, _CH2)], inbufs[par][1],
                             lsems[par]),
            pltpu.async_copy(tu_h.at[pl.ds(2 * off, 2 * _CH2)],
                             inbufs[par][2], lsems[par]),
        )

    pend = {0: startl(0, 0)}
    outpend = {}
    for s in range(_NS2):
        par = s % 2
        for d in pend.pop(s):
            d.wait()
        if s + 1 < _NS2:
            pend[s + 1] = startl(s + 1, (s + 1) % 2)
        if s >= 2:
            outpend.pop(s - 2).wait()
        rb, cb, ub = inbufs[par]
        ob = obufs[par]

        def grp(g, carry):
            go = g * 16
            r16 = rb[pl.ds(go, 16)]
            c16 = cb[pl.ds(go, 16)]
            u16 = ub[pl.ds(((go >> 6) << 7) + (go & 63) + 64, 16)]
            z = (plsc.load_gather(g1_v, [r16]) + plsc.load_gather(g2_v, [c16])
                 + u16 + bm)
            ob[pl.ds(go, 16)] = 1.0 / (1.0 + jnp.exp(-z))
            return carry

        lax.fori_loop(0, _CH2 // 16, grp, 0)
        outpend[s] = pltpu.async_copy(
            ob, out_h.at[pl.ds(base + s * _CH2, _CH2)], ssems[par])
    for s in (_NS2 - 2, _NS2 - 1):
        outpend.pop(s).wait()


_scc_call = pl.kernel(
    _scc_body,
    out_type=jax.ShapeDtypeStruct((_EP,), jnp.float32),
    mesh=plsc.VectorSubcoreMesh(core_axis_name="c", subcore_axis_name="s"),
    compiler_params=pltpu.CompilerParams(use_tc_tiling_on_sc=False,
                                         needs_layout_passes=False),
    scratch_types=[
        pltpu.VMEM((_NP,), jnp.float32), pltpu.VMEM((_NP,), jnp.float32),
        pltpu.VMEM((16,), jnp.float32),
        pltpu.VMEM((_CH2,), jnp.int32), pltpu.VMEM((_CH2,), jnp.int32),
        pltpu.VMEM((_CH2,), jnp.int32), pltpu.VMEM((_CH2,), jnp.int32),
        pltpu.VMEM((2 * _CH2,), jnp.float32),
        pltpu.VMEM((2 * _CH2,), jnp.float32),
        pltpu.VMEM((_CH2,), jnp.float32), pltpu.VMEM((_CH2,), jnp.float32),
        pltpu.SemaphoreType.DMA, pltpu.SemaphoreType.DMA,
        pltpu.SemaphoreType.DMA, pltpu.SemaphoreType.DMA,
    ],
)


def kernel(x, edge_index, edge_attr, W_src, att_src, att_dst, W_edge,
           att_edge, bias, W_mlp, b_mlp):
    f32 = jnp.float32
    xs = x[:, 0]
    row = edge_index[0]
    col = edge_index[1]
    c_s = jnp.sum(W_src[0] * att_src)
    c_d = jnp.sum(W_src[0] * att_dst)
    v_e = W_edge @ att_edge
    w1 = W_mlp[:_HID, 0]
    w2 = W_mlp[_HID:2 * _HID, 0]
    w3 = W_mlp[2 * _HID:, 0]

    p64 = jnp.concatenate(
        [jnp.kron(jnp.eye(64, dtype=f32), v_e[:, None]),
         jnp.kron(jnp.eye(64, dtype=f32), w3[:, None])], axis=1)
    ea64 = jnp.concatenate(
        [edge_attr.reshape(_E * 16),
         jnp.zeros((_ER * 1024 - _E * 16,), f32)]).reshape(_ER, 1024)
    tu = _edge_proj(ea64, p64).reshape(2 * _EP)

    pad_e = _EP - _E
    rowp = jnp.concatenate([row, jnp.full((pad_e,), _N, jnp.int32)])
    colp = jnp.concatenate([col, jnp.full((pad_e,), _N, jnp.int32)])
    xp = jnp.concatenate([xs, jnp.zeros((_NP - _N,), f32)])
    cs16 = jnp.full((16,), c_s, f32)
    cd16 = jnp.full((16,), c_d, f32)

    acc = _scb_call(rowp, colp, tu, xp, cs16, cd16)

    csd = jnp.reshape(c_s + c_d, (1, 1))
    wsb = jnp.stack([W_src[0], bias])
    wg = jnp.stack([w1, w2], axis=1)
    g1, g2 = _node_pass(csd, acc.reshape(2, 4, _NP), xp, wsb, wg)

    bm16 = jnp.full((16,), b_mlp[0], f32)
    outp = _scc_call(rowp, colp, tu, g1, g2, bm16)
    return outp[:_E][:, None]


# speedup vs baseline: 1.0974x; 1.0974x over previous
import jax
import jax.numpy as jnp
from jax import lax
from jax.experimental import pallas as pl
from jax.experimental.pallas import tpu as pltpu
from jax.experimental.pallas import tpu_sc as plsc

_N = 50000
_E = 800000
_HID = 32
_NP = 51200
_EW = _E // 32
_SCH = 4992
_NSC = 5
_TB = _NSC * _SCH
_CH2 = 1792
_NS2 = 14
_TC2 = 13 * _CH2
_CHT = _EW - _TC2
_NROWS = _NP // 16


def _edge_proj_body(ea_ref, p_ref, tu_ref):
    tu_ref[...] = jnp.dot(ea_ref[...], p_ref[...],
                          preferred_element_type=jnp.float32)


def _edge_proj(ea8, p):
    m = ea8.shape[0]
    blk = 800
    return pl.pallas_call(
        _edge_proj_body,
        grid=(m // blk,),
        in_specs=[pl.BlockSpec((blk, 128), lambda i: (i, 0)),
                  pl.BlockSpec((128, 16), lambda i: (0, 0))],
        out_specs=pl.BlockSpec((blk, 16), lambda i: (i, 0)),
        out_shape=jax.ShapeDtypeStruct((m, 16), jnp.float32),
    )(ea8, p)


def _scb_compute_group(rb, cb, tb, x_v, cs, cd, pat_t, go):
    r16 = rb[pl.ds(go, 16)]
    c16 = cb[pl.ds(go, 16)]
    t16 = plsc.load_gather(tb, [2 * go + pat_t])
    xr = plsc.load_gather(x_v, [r16])
    xc = plsc.load_gather(x_v, [c16])
    al = cs * xr + cd * xc + t16
    al = jnp.where(al >= 0.0, al, 0.2 * al)
    ex = jnp.exp(al)
    return r16, c16, t16, xr, ex


def _scb_body(row_h, col_h, tu_h, x_h, cs_h, cd_h, acc_h,
              x_v, r0, r1, c0, c1, t0, t1,
              exbA, exrbA, tvbA, idxbA, exbB, exrbB, tvbB, idxbB,
              ones_v, zbuf, rT, cT, tuT, tT, exT, exrT, oT, idxT,
              csv, cdv, acc0, acc1, acc2, acc3,
              sem0, sem1, semsA, semsB):
    cid = lax.axis_index("c")
    tid = lax.axis_index("s")
    base = (cid * 16 + tid) * _EW
    pltpu.sync_copy(x_h, x_v)
    pltpu.sync_copy(cs_h, csv)
    pltpu.sync_copy(cd_h, cdv)
    zeros16 = jnp.zeros((16,), jnp.float32)
    ones16 = jnp.ones((16,), jnp.float32)

    def zfill(i, carry):
        zbuf[pl.ds(i * 16, 16)] = zeros16
        return carry

    lax.fori_loop(0, _NROWS // 16, zfill, 0)
    for g in range(8):
        ones_v[pl.ds(g * 16, 16)] = ones16
    nbase = tid * _NROWS
    accs = (acc0, acc1, acc2, acc3)
    for a in accs:
        pltpu.sync_copy(zbuf, a.at[pl.ds(nbase, _NROWS)])
    plsc.subcore_barrier()

    cs = csv[...]
    cd = cdv[...]
    iota16 = lax.iota(jnp.int32, 16)
    pat_t = ((iota16 >> 3) << 4) + (iota16 & 7)
    bufs = ((r0, c0, t0), (r1, c1, t1))
    sets = ((tvbA, exbA, exrbA, idxbA, semsA),
            (tvbB, exbB, exrbB, idxbB, semsB))
    lsems = (sem0, sem1)

    def start(sidx, par):
        off = base + sidx * _SCH
        return (
            pltpu.async_copy(row_h.at[pl.ds(off, _SCH)], bufs[par][0],
                             lsems[par]),
            pltpu.async_copy(col_h.at[pl.ds(off, _SCH)], bufs[par][1],
                             lsems[par]),
            pltpu.async_copy(tu_h.at[pl.ds(2 * off, 2 * _SCH)], bufs[par][2],
                             lsems[par]),
        )

    def fill_and_fire(rb, cb, tb, o, bset):
        tvb, exb, exrb, idxb, ssem = bset
        for g in range(8):
            go = o + g * 16
            r16, c16, t16, xr, ex = _scb_compute_group(
                rb, cb, tb, x_v, cs, cd, pat_t, go)
            lo = g * 16
            tvb[pl.ds(lo, 16)] = t16
            exb[pl.ds(lo, 16)] = ex
            exrb[pl.ds(lo, 16)] = ex * xr
            idxb[pl.ds(lo, 16)] = c16
        return (
            pltpu.async_copy(ones_v, acc0.at[idxb], ssem, add=True),
            pltpu.async_copy(tvb, acc1.at[idxb], ssem, add=True),
            pltpu.async_copy(exb, acc2.at[idxb], ssem, add=True),
            pltpu.async_copy(exrb, acc3.at[idxb], ssem, add=True),
        )

    pend = {0: start(0, 0)}
    for s in range(_NSC):
        par = s % 2
        for d in pend.pop(s):
            d.wait()
        if s + 1 < _NSC:
            pend[s + 1] = start(s + 1, (s + 1) % 2)
        rb, cb, tb = bufs[par]

        def chunk2(j, carry):
            dsA = fill_and_fire(rb, cb, tb, j * 256, sets[0])
            dsB = fill_and_fire(rb, cb, tb, j * 256 + 128, sets[1])
            for d in dsA:
                d.wait()
            for d in dsB:
                d.wait()
            return carry

        lax.fori_loop(0, 19, chunk2, 0)
        for d in fill_and_fire(rb, cb, tb, 38 * 128, sets[0]):
            d.wait()

    dt = (
        pltpu.async_copy(row_h.at[pl.ds(base + _TB, 40)],
                         rT.at[pl.ds(0, 40)], sem0),
        pltpu.async_copy(col_h.at[pl.ds(base + _TB, 40)],
                         cT.at[pl.ds(0, 40)], sem0),
        pltpu.async_copy(tu_h.at[pl.ds(2 * (base + _TB), 80)],
                         tuT.at[pl.ds(0, 80)], sem0),
    )
    for d in dt:
        d.wait()
    m8 = iota16 < 8
    for g in range(3):
        go = g * 16
        r16 = rT[pl.ds(go, 16)]
        c16 = cT[pl.ds(go, 16)]
        t16 = plsc.load_gather(tuT, [2 * go + pat_t])
        if g == 2:
            r16 = jnp.where(m8, r16, 0)
            c16 = jnp.where(m8, c16, 0)
            t16 = jnp.where(m8, t16, 0.0)
        tT[pl.ds(go, 16)] = t16
        xr = plsc.load_gather(x_v, [r16])
        xc = plsc.load_gather(x_v, [c16])
        al = cs * xr + cd * xc + t16
        al = jnp.where(al >= 0.0, al, 0.2 * al)
        ex = jnp.exp(al)
        exr = ex * xr
        one = jnp.ones((16,), jnp.float32)
        idx = c16
        if g == 2:
            ex = jnp.where(m8, ex, 0.0)
            exr = jnp.where(m8, exr, 0.0)
            one = jnp.where(m8, one, 0.0)
            idx = jnp.where(m8, c16, _N)
        exT[pl.ds(go, 16)] = ex
        exrT[pl.ds(go, 16)] = exr
        oT[pl.ds(go, 16)] = one
        idxT[pl.ds(go, 16)] = idx
    dt2 = (
        pltpu.async_copy(oT, acc0.at[idxT], semsA, add=True),
        pltpu.async_copy(tT, acc1.at[idxT], semsA, add=True),
        pltpu.async_copy(exT, acc2.at[idxT], semsA, add=True),
        pltpu.async_copy(exrT, acc3.at[idxT], semsA, add=True),
    )
    for d in dt2:
        d.wait()

    plsc.subcore_barrier()
    for q in range(4):
        pltpu.sync_copy(accs[q].at[pl.ds(nbase, _NROWS)],
                        acc_h.at[pl.ds((cid * 4 + q) * _NP + nbase, _NROWS)])


_scb_call = pl.kernel(
    _scb_body,
    out_type=jax.ShapeDtypeStruct((8 * _NP,), jnp.float32),
    mesh=plsc.VectorSubcoreMesh(core_axis_name="c", subcore_axis_name="s"),
    compiler_params=pltpu.CompilerParams(use_tc_tiling_on_sc=False,
                                         needs_layout_passes=False),
    scratch_types=[
        pltpu.VMEM((_NP,), jnp.float32),
        pltpu.VMEM((_SCH,), jnp.int32), pltpu.VMEM((_SCH,), jnp.int32),
        pltpu.VMEM((_SCH,), jnp.int32), pltpu.VMEM((_SCH,), jnp.int32),
        pltpu.VMEM((2 * _SCH,), jnp.float32),
        pltpu.VMEM((2 * _SCH,), jnp.float32),
        pltpu.VMEM((128,), jnp.float32), pltpu.VMEM((128,), jnp.float32),
        pltpu.VMEM((128,), jnp.float32), pltpu.VMEM((128,), jnp.int32),
        pltpu.VMEM((128,), jnp.float32), pltpu.VMEM((128,), jnp.float32),
        pltpu.VMEM((128,), jnp.float32), pltpu.VMEM((128,), jnp.int32),
        pltpu.VMEM((128,), jnp.float32),
        pltpu.VMEM((_NROWS,), jnp.float32),
        pltpu.VMEM((48,), jnp.int32), pltpu.VMEM((48,), jnp.int32),
        pltpu.VMEM((96,), jnp.float32),
        pltpu.VMEM((48,), jnp.float32), pltpu.VMEM((48,), jnp.float32),
        pltpu.VMEM((48,), jnp.float32), pltpu.VMEM((48,), jnp.float32),
        pltpu.VMEM((48,), jnp.int32),
        pltpu.VMEM((16,), jnp.float32), pltpu.VMEM((16,), jnp.float32),
        pltpu.VMEM_SHARED((_NP,), jnp.float32),
        pltpu.VMEM_SHARED((_NP,), jnp.float32),
        pltpu.VMEM_SHARED((_NP,), jnp.float32),
        pltpu.VMEM_SHARED((_NP,), jnp.float32),
        pltpu.SemaphoreType.DMA, pltpu.SemaphoreType.DMA,
        pltpu.SemaphoreType.DMA, pltpu.SemaphoreType.DMA,
    ],
)


def _node_body(p_ref, acc_ref, x_ref, wsb_ref, wg_ref, g1_ref, g2_ref):
    csd = p_ref[0, 0]
    a = acc_ref[0] + acc_ref[1]
    deg = a[0]
    st = a[1]
    den = a[2]
    num = a[3]
    xs = x_ref[...]
    sl = st / jnp.maximum(deg, 1.0)
    als = csd * xs + sl
    als = jnp.where(als >= 0.0, als, 0.2 * als)
    exs = jnp.exp(als)
    s = (num + exs * xs) / (den + exs + 1e-16)
    h = jnp.maximum(s[:, None] * wsb_ref[0][None, :] + wsb_ref[1][None, :],
                    0.0)
    g = jnp.dot(h, wg_ref[...], preferred_element_type=jnp.float32)
    g1_ref[...] = g[:, 0]
    g2_ref[...] = g[:, 1]


def _node_pass(csd, acc, xp, wsb, wg):
    blk = 5120
    return pl.pallas_call(
        _node_body,
        grid=(_NP // blk,),
        in_specs=[
            pl.BlockSpec(memory_space=pltpu.SMEM),
            pl.BlockSpec((2, 4, blk), lambda i: (0, 0, i)),
            pl.BlockSpec((blk,), lambda i: (i,)),
            pl.BlockSpec((2, _HID), lambda i: (0, 0)),
            pl.BlockSpec((_HID, 2), lambda i: (0, 0)),
        ],
        out_specs=[pl.BlockSpec((blk,), lambda i: (i,)),
                   pl.BlockSpec((blk,), lambda i: (i,))],
        out_shape=[jax.ShapeDtypeStruct((_NP,), jnp.float32),
                   jax.ShapeDtypeStruct((_NP,), jnp.float32)],
    )(csd, acc, xp, wsb, wg)


def _scc_body(row_h, col_h, tu_h, g1_h, g2_h, bm_h, out_h,
              g1_v, g2_v, bmv, r0, r1, c0, c1, u0, u1, o0, o1,
              seml0, seml1, sems0, sems1):
    cid = lax.axis_index("c")
    tid = lax.axis_index("s")
    base = (cid * 16 + tid) * _EW
    pltpu.sync_copy(g1_h, g1_v)
    pltpu.sync_copy(g2_h, g2_v)
    pltpu.sync_copy(bm_h, bmv)
    bm = bmv[...]
    iota16 = lax.iota(jnp.int32, 16)
    pat_u = ((iota16 >> 3) << 4) + (iota16 & 7) + 8
    inbufs = ((r0, c0, u0), (r1, c1, u1))
    obufs = (o0, o1)
    lsems = (seml0, seml1)
    ssems = (sems0, sems1)

    def startl(sidx, par):
        off = base + sidx * _CH2
        sz = _CH2 if sidx < 13 else _CHT
        return (
            pltpu.async_copy(row_h.at[pl.ds(off, sz)],
                             inbufs[par][0].at[pl.ds(0, sz)], lsems[par]),
            pltpu.async_copy(col_h.at[pl.ds(off, sz)],
                             inbufs[par][1].at[pl.ds(0, sz)], lsems[par]),
            pltpu.async_copy(tu_h.at[pl.ds(2 * off, 2 * sz)],
                             inbufs[par][2].at[pl.ds(0, 2 * sz)], lsems[par]),
        )

    pend = {0: startl(0, 0)}
    outpend = {}
    for s in range(_NS2):
        par = s % 2
        for d in pend.pop(s):
            d.wait()
        if s + 1 < _NS2:
            pend[s + 1] = startl(s + 1, (s + 1) % 2)
        if s >= 2:
            outpend.pop(s - 2).wait()
        rb, cb, ub = inbufs[par]
        ob = obufs[par]

        def grp(g, carry):
            go = g * 16
            r16 = rb[pl.ds(go, 16)]
            c16 = cb[pl.ds(go, 16)]
            u16 = plsc.load_gather(ub, [2 * go + pat_u])
            z = (plsc.load_gather(g1_v, [r16]) + plsc.load_gather(g2_v, [c16])
                 + u16 + bm)
            ob[pl.ds(go, 16)] = 1.0 / (1.0 + jnp.exp(-z))
            return carry

        lax.fori_loop(0, 112 if s < 13 else 106, grp, 0)
        if s == 13:
            m8 = iota16 < 8
            go = 106 * 16
            r16 = jnp.where(m8, rb[pl.ds(go, 16)], 0)
            c16 = jnp.where(m8, cb[pl.ds(go, 16)], 0)
            u16 = plsc.load_gather(ub, [2 * go + pat_u])
            z = (plsc.load_gather(g1_v, [r16]) + plsc.load_gather(g2_v, [c16])
                 + u16 + bm)
            ob[pl.ds(go, 16)] = 1.0 / (1.0 + jnp.exp(-z))
        sz = _CH2 if s < 13 else _CHT
        outpend[s] = pltpu.async_copy(
            ob.at[pl.ds(0, sz)], out_h.at[pl.ds(base + s * _CH2, sz)],
            ssems[par])
    for s in (_NS2 - 2, _NS2 - 1):
        outpend.pop(s).wait()


_scc_call = pl.kernel(
    _scc_body,
    out_type=jax.ShapeDtypeStruct((_E,), jnp.float32),
    mesh=plsc.VectorSubcoreMesh(core_axis_name="c", subcore_axis_name="s"),
    compiler_params=pltpu.CompilerParams(use_tc_tiling_on_sc=False,
                                         needs_layout_passes=False),
    scratch_types=[
        pltpu.VMEM((_NP,), jnp.float32), pltpu.VMEM((_NP,), jnp.float32),
        pltpu.VMEM((16,), jnp.float32),
        pltpu.VMEM((_CH2,), jnp.int32), pltpu.VMEM((_CH2,), jnp.int32),
        pltpu.VMEM((_CH2,), jnp.int32), pltpu.VMEM((_CH2,), jnp.int32),
        pltpu.VMEM((2 * _CH2,), jnp.float32),
        pltpu.VMEM((2 * _CH2,), jnp.float32),
        pltpu.VMEM((_CH2,), jnp.float32), pltpu.VMEM((_CH2,), jnp.float32),
        pltpu.SemaphoreType.DMA, pltpu.SemaphoreType.DMA,
        pltpu.SemaphoreType.DMA, pltpu.SemaphoreType.DMA,
    ],
)


def kernel(x, edge_index, edge_attr, W_src, att_src, att_dst, W_edge,
           att_edge, bias, W_mlp, b_mlp):
    f32 = jnp.float32
    xs = x[:, 0]
    row = edge_index[0]
    col = edge_index[1]
    c_s = jnp.sum(W_src[0] * att_src)
    c_d = jnp.sum(W_src[0] * att_dst)
    v_e = W_edge @ att_edge
    w1 = W_mlp[:_HID, 0]
    w2 = W_mlp[_HID:2 * _HID, 0]
    w3 = W_mlp[2 * _HID:, 0]

    p = jnp.concatenate([jnp.kron(jnp.eye(8, dtype=f32), v_e[:, None]),
                         jnp.kron(jnp.eye(8, dtype=f32), w3[:, None])], axis=1)
    tu = _edge_proj(edge_attr.reshape(_E // 8, 128), p).reshape(2 * _E)

    xp = jnp.concatenate([xs, jnp.zeros((_NP - _N,), f32)])
    cs16 = jnp.full((16,), c_s, f32)
    cd16 = jnp.full((16,), c_d, f32)

    acc = _scb_call(row, col, tu, xp, cs16, cd16)

    csd = jnp.reshape(c_s + c_d, (1, 1))
    wsb = jnp.stack([W_src[0], bias])
    wg = jnp.stack([w1, w2], axis=1)
    g1, g2 = _node_pass(csd, acc.reshape(2, 4, _NP), xp, wsb, wg)

    bm16 = jnp.full((16,), b_mlp[0], f32)
    outp = _scc_call(row, col, tu, g1, g2, bm16)
    return outp.reshape(_E, 1)
